# baseline (speedup 1.0000x reference)
import functools

import jax
import jax.numpy as jnp
from jax import lax
from jax.experimental import pallas as pl
from jax.experimental.pallas import tpu as pltpu
from jax.experimental.pallas import tpu_sc as plsc

NC = 2
NS = 16
LANES = 128


def _round_up(a, b):
    return (a + b - 1) // b * b


def _sc_deg(dst3, np_, cpw):
    rps = np_ // NS

    mesh = plsc.VectorSubcoreMesh(core_axis_name="c", subcore_axis_name="s")

    @functools.partial(
        pl.kernel,
        out_type=jax.ShapeDtypeStruct((NC * np_,), jnp.float32),
        mesh=mesh,
        scratch_types=[
            pltpu.VMEM((cpw, LANES), jnp.int32),
            pltpu.VMEM((LANES,), jnp.float32),
            pltpu.VMEM((rps,), jnp.float32),
            pltpu.VMEM_SHARED((np_,), jnp.float32),
            pltpu.SemaphoreType.DMA,
        ],
    )
    def deg_kernel(dst_hbm, degp_hbm, dstv, onesv, zrow, degsh, sem):
        core = lax.axis_index("c")
        sub = lax.axis_index("s")
        wid = sub * NC + core
        pltpu.sync_copy(dst_hbm.at[wid], dstv)

        for k in range(LANES // 16):
            onesv[pl.ds(k * 16, 16)] = jnp.full((16,), 1.0, jnp.float32)

        def zbody(i, _):
            zrow[pl.ds(pl.multiple_of(i * 16, 16), 16)] = jnp.zeros(
                (16,), jnp.float32)
            return 0

        lax.fori_loop(0, rps // 16, zbody, 0)

        base = pl.multiple_of(sub * rps, 128)
        pltpu.sync_copy(zrow, degsh.at[pl.ds(base, rps)])
        plsc.subcore_barrier()

        def ebody(j, _):
            @pl.when(j >= 8)
            def _():
                pltpu.make_async_copy(onesv, degsh.at[dstv.at[j]], sem).wait()

            pltpu.make_async_copy(
                onesv, degsh.at[dstv.at[j]], sem).start(add=True)
            return 0

        lax.fori_loop(0, cpw, ebody, 0)

        def dbody(j, _):
            pltpu.make_async_copy(onesv, degsh.at[dstv.at[j]], sem).wait()
            return 0

        lax.fori_loop(0, min(8, cpw), dbody, 0)
        plsc.subcore_barrier()

        obase = pl.multiple_of(core * np_ + sub * rps, 128)
        pltpu.sync_copy(degsh.at[pl.ds(base, rps)], degp_hbm.at[pl.ds(obase, rps)])

    return deg_kernel(dst3)


def _sc_edges(src3, dst3, xs, dis, np_, cpw, fin):
    rps = np_ // NS
    NR = 2
    ND = 4
    NI = 8
    assert cpw % NI == 0 and cpw >= NI

    mesh = plsc.VectorSubcoreMesh(core_axis_name="c", subcore_axis_name="s")

    @functools.partial(
        pl.kernel,
        out_type=(
            jax.ShapeDtypeStruct((NC * np_, fin), jnp.float32),
            jax.ShapeDtypeStruct((NC * np_,), jnp.float32),
        ),
        mesh=mesh,
        scratch_types=[
            pltpu.VMEM((NI, LANES), jnp.int32),
            pltpu.VMEM((NI, LANES), jnp.int32),
            pltpu.VMEM((NR, LANES, fin), jnp.float32),
            pltpu.VMEM((ND, LANES), jnp.float32),
            pltpu.VMEM((rps,), jnp.float32),
            pltpu.VMEM_SHARED((np_, fin), jnp.float32),
            pltpu.VMEM_SHARED((np_,), jnp.float32),
        ]
        + [pltpu.SemaphoreType.DMA] * (2 * NI + NR + 2 * ND),
    )
    def edge_kernel(src_hbm, dst_hbm, xs_hbm, dis_hbm, tp_hbm, cp_hbm,
                    srcv, dstv, rows, dvals, zrow, tsh, csh, *sems):
        isems = sems[:NI]
        isemd = sems[NI:2 * NI]
        gsem = sems[2 * NI:2 * NI + NR]
        dsem = sems[2 * NI + NR:2 * NI + NR + ND]
        csem = sems[2 * NI + NR + ND:]
        core = lax.axis_index("c")
        sub = lax.axis_index("s")
        wid = sub * NC + core

        def idx_start(j, slot):
            pltpu.async_copy(src_hbm.at[wid, j], srcv.at[slot], isems[slot])
            pltpu.async_copy(dst_hbm.at[wid, j], dstv.at[slot], isemd[slot])

        def idx_wait(j, slot):
            pltpu.make_async_copy(
                src_hbm.at[wid, j], srcv.at[slot], isems[slot]).wait()
            pltpu.make_async_copy(
                dst_hbm.at[wid, j], dstv.at[slot], isemd[slot]).wait()

        def rows_start(slot_i, slot_r):
            pltpu.async_copy(
                xs_hbm.at[srcv.at[slot_i]], rows.at[slot_r], gsem[slot_r])

        def rows_wait(slot_i, slot_r):
            pltpu.make_async_copy(
                xs_hbm.at[srcv.at[slot_i]], rows.at[slot_r],
                gsem[slot_r]).wait()

        def dis_start(slot_i, slot_d):
            pltpu.async_copy(
                dis_hbm.at[dstv.at[slot_i]], dvals.at[slot_d], dsem[slot_d])

        def dis_wait(slot_i, slot_d):
            pltpu.make_async_copy(
                dis_hbm.at[dstv.at[slot_i]], dvals.at[slot_d],
                dsem[slot_d]).wait()

        def csc_start(slot_i, slot_d):
            pltpu.make_async_copy(
                dvals.at[slot_d], csh.at[srcv.at[slot_i]],
                csem[slot_d]).start(add=True)

        def csc_wait(slot_i, slot_d):
            pltpu.make_async_copy(
                dvals.at[slot_d], csh.at[srcv.at[slot_i]],
                csem[slot_d]).wait()

        def zr(i, _):
            for k in range(fin // 16):
                rows[0, i, pl.ds(k * 16, 16)] = jnp.zeros((16,), jnp.float32)
            return 0

        lax.fori_loop(0, LANES, zr, 0)

        def zc(i, _):
            zrow[pl.ds(pl.multiple_of(i * 16, 16), 16)] = jnp.zeros(
                (16,), jnp.float32)
            return 0

        lax.fori_loop(0, rps // 16, zc, 0)

        base = pl.multiple_of(sub * rps, 128)
        for k in range(rps // LANES):
            pltpu.sync_copy(rows.at[0], tsh.at[pl.ds(base + k * LANES, LANES)])
        pltpu.sync_copy(zrow, csh.at[pl.ds(base, rps)])

        for j in range(4):
            idx_start(j, j)
        idx_wait(0, 0)
        idx_wait(1, 1)
        rows_start(0, 0)
        dis_start(0, 0)
        dis_start(1, 1)
        plsc.subcore_barrier()

        def ebody(i, _):
            for b in range(NI):
                j = i * NI + b

                @pl.when(j + 4 < cpw)
                def _():
                    idx_start(j + 4, (b + 4) % NI)

                @pl.when(j + 2 < cpw)
                def _():
                    idx_wait(j + 2, (b + 2) % NI)

                @pl.when(j + 1 < cpw)
                def _():
                    rows_start((b + 1) % NI, (b + 1) % NR)

                rows_wait(b, b % NR)
                pltpu.sync_copy(rows.at[b % NR], tsh.at[dstv.at[b]], add=True)

                dis_wait(b, b % ND)
                csc_start(b, b % ND)

                @pl.when(j >= 2)
                def _():
                    csc_wait(b, (b + 2) % ND)

                @pl.when(j + 2 < cpw)
                def _():
                    dis_start((b + 2) % NI, (b + 2) % ND)
            return 0

        lax.fori_loop(0, cpw // NI, ebody, 0)
        csc_wait((cpw - 2) % NI, (cpw - 2) % ND)
        csc_wait((cpw - 1) % NI, (cpw - 1) % ND)
        plsc.subcore_barrier()

        obase = pl.multiple_of(core * np_ + sub * rps, 128)
        for k in range(rps // LANES):
            pltpu.sync_copy(tsh.at[pl.ds(base + k * LANES, LANES)],
                            tp_hbm.at[pl.ds(obase + k * LANES, LANES)])
        pltpu.sync_copy(csh.at[pl.ds(base, rps)], cp_hbm.at[pl.ds(obase, rps)])

    return edge_kernel(src3, dst3, xs, dis)


def _tc_matmul(x_pad, w1, np_, fin, bs):
    def body(x_ref, w_ref, o_ref):
        o_ref[:] = jnp.dot(x_ref[:], w_ref[:],
                           preferred_element_type=jnp.float32)

    return pl.pallas_call(
        body,
        grid=(np_ // bs,),
        in_specs=[
            pl.BlockSpec((bs, fin), lambda i: (i, 0)),
            pl.BlockSpec((fin, fin), lambda i: (0, 0)),
        ],
        out_specs=pl.BlockSpec((bs, fin), lambda i: (i, 0)),
        out_shape=jax.ShapeDtypeStruct((np_, fin), jnp.float32),
    )(x_pad, w1)


def _tc_prep(xw, degp, n, np_, fin, bs):
    def body(xw_ref, degp_ref, xs_ref, dis_ref):
        i = pl.program_id(0)
        dp = degp_ref[:]
        degsum = 1.0 + dp[0] + dp[1]
        rows = lax.broadcasted_iota(jnp.int32, (bs, 1), 0) + i * bs
        dis = jnp.where(rows < n, lax.rsqrt(degsum), 0.0)
        xs_ref[:] = xw_ref[:] * dis
        dis_ref[:] = dis

    return pl.pallas_call(
        body,
        grid=(np_ // bs,),
        in_specs=[
            pl.BlockSpec((bs, fin), lambda i: (i, 0)),
            pl.BlockSpec((NC, bs, 1), lambda i: (0, i, 0)),
        ],
        out_specs=[
            pl.BlockSpec((bs, fin), lambda i: (i, 0)),
            pl.BlockSpec((bs, 1), lambda i: (i, 0)),
        ],
        out_shape=(
            jax.ShapeDtypeStruct((np_, fin), jnp.float32),
            jax.ShapeDtypeStruct((np_, 1), jnp.float32),
        ),
    )(xw, degp)


def _tc_gsum(tp, xs, dis, cp, b1, np_, fin, bs):
    nblk = np_ // bs

    def body(tp_ref, xs_ref, dis_ref, cp_ref, b1_ref, g_ref):
        tp2 = tp_ref[:]
        t = tp2[0] + tp2[1]
        dis = dis_ref[:]
        cp2 = cp_ref[:]
        h = jnp.maximum(dis * (t + xs_ref[:]) + b1_ref[:], 0.0)
        wv = dis * (cp2[0] + cp2[1] + dis)
        g = lax.dot_general(wv, h, (((0,), (0,)), ((), ())),
                            preferred_element_type=jnp.float32)
        g_ref[:] = g.reshape(g_ref.shape)

    return pl.pallas_call(
        body,
        grid=(nblk,),
        in_specs=[
            pl.BlockSpec((NC, bs, fin), lambda i: (0, i, 0)),
            pl.BlockSpec((bs, fin), lambda i: (i, 0)),
            pl.BlockSpec((bs, 1), lambda i: (i, 0)),
            pl.BlockSpec((NC, bs, 1), lambda i: (0, i, 0)),
            pl.BlockSpec((1, fin), lambda i: (0, 0)),
        ],
        out_specs=pl.BlockSpec((1, 1, fin), lambda i: (i, 0, 0)),
        out_shape=jax.ShapeDtypeStruct((nblk, 1, fin), jnp.float32),
    )(tp, xs, dis, cp, b1)


def _tc_heads(gparts, wmu, bmu, wls, bls, n, fin, fout, bs):
    nblk = n // bs
    inv_n = 1.0 / n

    def body(g_ref, wmu_ref, bmu_ref, wls_ref, bls_ref, omu_ref, ols_ref):
        g = jnp.sum(g_ref[:], axis=0, keepdims=True) * inv_n
        mu = jnp.dot(g, wmu_ref[:], preferred_element_type=jnp.float32) \
            + bmu_ref[:]
        ls = jnp.dot(g, wls_ref[:], preferred_element_type=jnp.float32) \
            + bls_ref[:]
        omu_ref[:] = jnp.broadcast_to(mu, (bs, mu.shape[1]))
        ols_ref[:] = jnp.broadcast_to(ls, (bs, ls.shape[1]))

    nparts = gparts.shape[0]
    return pl.pallas_call(
        body,
        grid=(nblk,),
        in_specs=[
            pl.BlockSpec((nparts, fin), lambda i: (0, 0)),
            pl.BlockSpec((fin, fout), lambda i: (0, 0)),
            pl.BlockSpec((1, fout), lambda i: (0, 0)),
            pl.BlockSpec((fin, fout), lambda i: (0, 0)),
            pl.BlockSpec((1, fout), lambda i: (0, 0)),
        ],
        out_specs=[
            pl.BlockSpec((bs, fout), lambda i: (i, 0)),
            pl.BlockSpec((bs, fout), lambda i: (i, 0)),
        ],
        out_shape=(
            jax.ShapeDtypeStruct((n, fout), jnp.float32),
            jax.ShapeDtypeStruct((n, fout), jnp.float32),
        ),
    )(gparts, wmu, bmu, wls, bls)


def kernel(x, edge_index, W1, b1, Wmu, bmu, Wls, bls):
    n, fin = x.shape
    e = edge_index.shape[1]
    fout = Wmu.shape[1]
    nw = NC * NS

    np_ = _round_up(n + 1, NS * LANES)
    ep = _round_up(e, nw * LANES * 8)
    cpw = ep // (nw * LANES)

    src = edge_index[0]
    dst = edge_index[1]
    pad_e = ep - e
    src_p = jnp.concatenate(
        [src, jnp.zeros((pad_e,), jnp.int32)]).reshape(nw, cpw, LANES)
    dst_p = jnp.concatenate(
        [dst, jnp.full((pad_e,), n, jnp.int32)]).reshape(nw, cpw, LANES)

    x_pad = jnp.pad(x, ((0, np_ - n), (0, 0)))

    degp = _sc_deg(dst_p, np_, cpw)
    xw = _tc_matmul(x_pad, W1, np_, fin, 1024)

    degp2 = degp.reshape(NC, np_, 1)
    xs, dis2 = _tc_prep(xw, degp2, n, np_, fin, 1024)

    tp, cp = _sc_edges(src_p, dst_p, xs, dis2.reshape(np_), np_, cpw, fin)

    gparts = _tc_gsum(tp.reshape(NC, np_, fin), xs, dis2,
                      cp.reshape(NC, np_, 1), b1.reshape(1, fin),
                      np_, fin, 1024).reshape(-1, fin)

    out_mu, out_ls = _tc_heads(gparts, Wmu, bmu.reshape(1, fout),
                               Wls, bls.reshape(1, fout), n, fin, fout, 1000)
    return (out_mu, out_ls)


# speedup vs baseline: 1.0780x; 1.0780x over previous
import functools

import jax
import jax.numpy as jnp
from jax import lax
from jax.experimental import pallas as pl
from jax.experimental.pallas import tpu as pltpu
from jax.experimental.pallas import tpu_sc as plsc

NC = 2
NS = 16
LANES = 128


def _round_up(a, b):
    return (a + b - 1) // b * b


def _sc_deg(dst3, np_, cpw):
    rps = np_ // NS

    mesh = plsc.VectorSubcoreMesh(core_axis_name="c", subcore_axis_name="s")

    @functools.partial(
        pl.kernel,
        out_type=jax.ShapeDtypeStruct((NC * np_,), jnp.float32),
        mesh=mesh,
        scratch_types=[
            pltpu.VMEM((cpw, LANES), jnp.int32),
            pltpu.VMEM((LANES,), jnp.float32),
            pltpu.VMEM((rps,), jnp.float32),
            pltpu.VMEM_SHARED((np_,), jnp.float32),
            pltpu.SemaphoreType.DMA,
        ],
    )
    def deg_kernel(dst_hbm, degp_hbm, dstv, onesv, zrow, degsh, sem):
        core = lax.axis_index("c")
        sub = lax.axis_index("s")
        wid = sub * NC + core
        pltpu.sync_copy(dst_hbm.at[wid], dstv)

        for k in range(LANES // 16):
            onesv[pl.ds(k * 16, 16)] = jnp.full((16,), 1.0, jnp.float32)

        def zbody(i, _):
            zrow[pl.ds(pl.multiple_of(i * 16, 16), 16)] = jnp.zeros(
                (16,), jnp.float32)
            return 0

        lax.fori_loop(0, rps // 16, zbody, 0)

        base = pl.multiple_of(sub * rps, 128)
        pltpu.sync_copy(zrow, degsh.at[pl.ds(base, rps)])
        plsc.subcore_barrier()

        def ebody(j, _):
            @pl.when(j >= 8)
            def _():
                pltpu.make_async_copy(onesv, degsh.at[dstv.at[j]], sem).wait()

            pltpu.make_async_copy(
                onesv, degsh.at[dstv.at[j]], sem).start(add=True)
            return 0

        lax.fori_loop(0, cpw, ebody, 0)

        def dbody(j, _):
            pltpu.make_async_copy(onesv, degsh.at[dstv.at[j]], sem).wait()
            return 0

        lax.fori_loop(0, min(8, cpw), dbody, 0)
        plsc.subcore_barrier()

        obase = pl.multiple_of(core * np_ + sub * rps, 128)
        pltpu.sync_copy(degsh.at[pl.ds(base, rps)], degp_hbm.at[pl.ds(obase, rps)])

    return deg_kernel(dst3)


def _sc_edges(src3, dst3, xs, dis, np_, cpw, fin):
    rps = np_ // NS
    NR = 2
    ND = 4
    NI = 8
    assert cpw % NI == 0 and cpw >= NI

    mesh = plsc.VectorSubcoreMesh(
        core_axis_name="c", subcore_axis_name="s", num_cores=1)

    @functools.partial(
        pl.kernel,
        out_type=(
            jax.ShapeDtypeStruct((np_, fin), jnp.float32),
            jax.ShapeDtypeStruct((np_,), jnp.float32),
        ),
        mesh=mesh,
        scratch_types=[
            pltpu.VMEM((NI, LANES), jnp.int32),
            pltpu.VMEM((NI, LANES), jnp.int32),
            pltpu.VMEM((NR, LANES, fin), jnp.float32),
            pltpu.VMEM((ND, LANES), jnp.float32),
            pltpu.VMEM((rps,), jnp.float32),
            pltpu.VMEM_SHARED((np_, fin), jnp.float32),
            pltpu.VMEM_SHARED((np_,), jnp.float32),
        ]
        + [pltpu.SemaphoreType.DMA] * (2 * NI + NR + 2 * ND),
    )
    def edge_kernel(src_hbm, dst_hbm, xs_hbm, dis_hbm, tp_hbm, cp_hbm,
                    srcv, dstv, rows, dvals, zrow, tsh, csh, *sems):
        isems = sems[:NI]
        isemd = sems[NI:2 * NI]
        gsem = sems[2 * NI:2 * NI + NR]
        dsem = sems[2 * NI + NR:2 * NI + NR + ND]
        csem = sems[2 * NI + NR + ND:]
        sub = lax.axis_index("s")
        wid = sub

        def idx_start(j, slot):
            pltpu.async_copy(src_hbm.at[wid, j], srcv.at[slot], isems[slot])
            pltpu.async_copy(dst_hbm.at[wid, j], dstv.at[slot], isemd[slot])

        def idx_wait(j, slot):
            pltpu.make_async_copy(
                src_hbm.at[wid, j], srcv.at[slot], isems[slot]).wait()
            pltpu.make_async_copy(
                dst_hbm.at[wid, j], dstv.at[slot], isemd[slot]).wait()

        def rows_start(slot_i, slot_r):
            pltpu.async_copy(
                xs_hbm.at[srcv.at[slot_i]], rows.at[slot_r], gsem[slot_r])

        def rows_wait(slot_i, slot_r):
            pltpu.make_async_copy(
                xs_hbm.at[srcv.at[slot_i]], rows.at[slot_r],
                gsem[slot_r]).wait()

        def dis_start(slot_i, slot_d):
            pltpu.async_copy(
                dis_hbm.at[dstv.at[slot_i]], dvals.at[slot_d], dsem[slot_d])

        def dis_wait(slot_i, slot_d):
            pltpu.make_async_copy(
                dis_hbm.at[dstv.at[slot_i]], dvals.at[slot_d],
                dsem[slot_d]).wait()

        def csc_start(slot_i, slot_d):
            pltpu.make_async_copy(
                dvals.at[slot_d], csh.at[srcv.at[slot_i]],
                csem[slot_d]).start(add=True)

        def csc_wait(slot_i, slot_d):
            pltpu.make_async_copy(
                dvals.at[slot_d], csh.at[srcv.at[slot_i]],
                csem[slot_d]).wait()

        def zr(i, _):
            for k in range(fin // 16):
                rows[0, i, pl.ds(k * 16, 16)] = jnp.zeros((16,), jnp.float32)
            return 0

        lax.fori_loop(0, LANES, zr, 0)

        def zc(i, _):
            zrow[pl.ds(pl.multiple_of(i * 16, 16), 16)] = jnp.zeros(
                (16,), jnp.float32)
            return 0

        lax.fori_loop(0, rps // 16, zc, 0)

        base = pl.multiple_of(sub * rps, 128)
        for k in range(rps // LANES):
            pltpu.sync_copy(rows.at[0], tsh.at[pl.ds(base + k * LANES, LANES)])
        pltpu.sync_copy(zrow, csh.at[pl.ds(base, rps)])

        for j in range(4):
            idx_start(j, j)
        idx_wait(0, 0)
        idx_wait(1, 1)
        rows_start(0, 0)
        dis_start(0, 0)
        dis_start(1, 1)
        plsc.subcore_barrier()

        def ebody(i, _):
            for b in range(NI):
                j = i * NI + b

                @pl.when(j + 4 < cpw)
                def _():
                    idx_start(j + 4, (b + 4) % NI)

                @pl.when(j + 2 < cpw)
                def _():
                    idx_wait(j + 2, (b + 2) % NI)

                @pl.when(j + 1 < cpw)
                def _():
                    rows_start((b + 1) % NI, (b + 1) % NR)

                rows_wait(b, b % NR)
                pltpu.sync_copy(rows.at[b % NR], tsh.at[dstv.at[b]], add=True)

                dis_wait(b, b % ND)
                csc_start(b, b % ND)

                @pl.when(j >= 2)
                def _():
                    csc_wait(b, (b + 2) % ND)

                @pl.when(j + 2 < cpw)
                def _():
                    dis_start((b + 2) % NI, (b + 2) % ND)
            return 0

        lax.fori_loop(0, cpw // NI, ebody, 0)
        csc_wait((cpw - 2) % NI, (cpw - 2) % ND)
        csc_wait((cpw - 1) % NI, (cpw - 1) % ND)
        plsc.subcore_barrier()

        for k in range(rps // LANES):
            pltpu.sync_copy(tsh.at[pl.ds(base + k * LANES, LANES)],
                            tp_hbm.at[pl.ds(base + k * LANES, LANES)])
        pltpu.sync_copy(csh.at[pl.ds(base, rps)], cp_hbm.at[pl.ds(base, rps)])

    return edge_kernel(src3, dst3, xs, dis)


def _tc_matmul(x_pad, w1, np_, fin, bs):
    def body(x_ref, w_ref, o_ref):
        o_ref[:] = jnp.dot(x_ref[:], w_ref[:],
                           preferred_element_type=jnp.float32)

    return pl.pallas_call(
        body,
        grid=(np_ // bs,),
        in_specs=[
            pl.BlockSpec((bs, fin), lambda i: (i, 0)),
            pl.BlockSpec((fin, fin), lambda i: (0, 0)),
        ],
        out_specs=pl.BlockSpec((bs, fin), lambda i: (i, 0)),
        out_shape=jax.ShapeDtypeStruct((np_, fin), jnp.float32),
    )(x_pad, w1)


def _tc_prep(xw, degp, n, np_, fin, bs):
    def body(xw_ref, degp_ref, xs_ref, dis_ref):
        i = pl.program_id(0)
        dp = degp_ref[:]
        degsum = 1.0 + dp[0] + dp[1]
        rows = lax.broadcasted_iota(jnp.int32, (bs, 1), 0) + i * bs
        dis = jnp.where(rows < n, lax.rsqrt(degsum), 0.0)
        xs_ref[:] = xw_ref[:] * dis
        dis_ref[:] = dis

    return pl.pallas_call(
        body,
        grid=(np_ // bs,),
        in_specs=[
            pl.BlockSpec((bs, fin), lambda i: (i, 0)),
            pl.BlockSpec((NC, bs, 1), lambda i: (0, i, 0)),
        ],
        out_specs=[
            pl.BlockSpec((bs, fin), lambda i: (i, 0)),
            pl.BlockSpec((bs, 1), lambda i: (i, 0)),
        ],
        out_shape=(
            jax.ShapeDtypeStruct((np_, fin), jnp.float32),
            jax.ShapeDtypeStruct((np_, 1), jnp.float32),
        ),
    )(xw, degp)


def _tc_gsum(tp0, tp1, xs, dis, cp0, cp1, b1, np_, fin, bs):
    nblk = np_ // bs

    def body(tp0_ref, tp1_ref, xs_ref, dis_ref, cp0_ref, cp1_ref, b1_ref,
             g_ref):
        t = tp0_ref[:] + tp1_ref[:]
        dis = dis_ref[:]
        h = jnp.maximum(dis * (t + xs_ref[:]) + b1_ref[:], 0.0)
        wv = dis * (cp0_ref[:] + cp1_ref[:] + dis)
        g = lax.dot_general(wv, h, (((0,), (0,)), ((), ())),
                            preferred_element_type=jnp.float32)
        g_ref[:] = g.reshape(g_ref.shape)

    return pl.pallas_call(
        body,
        grid=(nblk,),
        in_specs=[
            pl.BlockSpec((bs, fin), lambda i: (i, 0)),
            pl.BlockSpec((bs, fin), lambda i: (i, 0)),
            pl.BlockSpec((bs, fin), lambda i: (i, 0)),
            pl.BlockSpec((bs, 1), lambda i: (i, 0)),
            pl.BlockSpec((bs, 1), lambda i: (i, 0)),
            pl.BlockSpec((bs, 1), lambda i: (i, 0)),
            pl.BlockSpec((1, fin), lambda i: (0, 0)),
        ],
        out_specs=pl.BlockSpec((1, 1, fin), lambda i: (i, 0, 0)),
        out_shape=jax.ShapeDtypeStruct((nblk, 1, fin), jnp.float32),
    )(tp0, tp1, xs, dis, cp0, cp1, b1)


def _tc_heads(gparts, wmu, bmu, wls, bls, n, fin, fout, bs):
    nblk = n // bs
    inv_n = 1.0 / n

    def body(g_ref, wmu_ref, bmu_ref, wls_ref, bls_ref, omu_ref, ols_ref):
        g = jnp.sum(g_ref[:], axis=0, keepdims=True) * inv_n
        mu = jnp.dot(g, wmu_ref[:], preferred_element_type=jnp.float32) \
            + bmu_ref[:]
        ls = jnp.dot(g, wls_ref[:], preferred_element_type=jnp.float32) \
            + bls_ref[:]
        omu_ref[:] = jnp.broadcast_to(mu, (bs, mu.shape[1]))
        ols_ref[:] = jnp.broadcast_to(ls, (bs, ls.shape[1]))

    nparts = gparts.shape[0]
    return pl.pallas_call(
        body,
        grid=(nblk,),
        in_specs=[
            pl.BlockSpec((nparts, fin), lambda i: (0, 0)),
            pl.BlockSpec((fin, fout), lambda i: (0, 0)),
            pl.BlockSpec((1, fout), lambda i: (0, 0)),
            pl.BlockSpec((fin, fout), lambda i: (0, 0)),
            pl.BlockSpec((1, fout), lambda i: (0, 0)),
        ],
        out_specs=[
            pl.BlockSpec((bs, fout), lambda i: (i, 0)),
            pl.BlockSpec((bs, fout), lambda i: (i, 0)),
        ],
        out_shape=(
            jax.ShapeDtypeStruct((n, fout), jnp.float32),
            jax.ShapeDtypeStruct((n, fout), jnp.float32),
        ),
    )(gparts, wmu, bmu, wls, bls)


def kernel(x, edge_index, W1, b1, Wmu, bmu, Wls, bls):
    n, fin = x.shape
    e = edge_index.shape[1]
    fout = Wmu.shape[1]
    nw = NC * NS

    np_ = _round_up(n + 1, NS * LANES)
    ep = _round_up(e, nw * LANES * 8)
    cpw = ep // (nw * LANES)

    src = edge_index[0]
    dst = edge_index[1]
    pad_e = ep - e
    src_p = jnp.concatenate(
        [src, jnp.zeros((pad_e,), jnp.int32)]).reshape(2, NS, cpw, LANES)
    dst_p = jnp.concatenate(
        [dst, jnp.full((pad_e,), n, jnp.int32)]).reshape(2, NS, cpw, LANES)

    x_pad = jnp.pad(x, ((0, np_ - n), (0, 0)))

    degp = _sc_deg(dst_p.reshape(nw, cpw, LANES), np_, cpw)
    xw = _tc_matmul(x_pad, W1, np_, fin, 1024)

    degp2 = degp.reshape(NC, np_, 1)
    xs, dis2 = _tc_prep(xw, degp2, n, np_, fin, 1024)

    dis1 = dis2.reshape(np_)
    tp0, cp0 = _sc_edges(src_p[0], dst_p[0], xs, dis1, np_, cpw, fin)
    tp1, cp1 = _sc_edges(src_p[1], dst_p[1], xs, dis1, np_, cpw, fin)

    gparts = _tc_gsum(tp0, tp1, xs, dis2,
                      cp0.reshape(np_, 1), cp1.reshape(np_, 1),
                      b1.reshape(1, fin),
                      np_, fin, 1024).reshape(-1, fin)

    out_mu, out_ls = _tc_heads(gparts, Wmu, bmu.reshape(1, fout),
                               Wls, bls.reshape(1, fout), n, fin, fout, 1000)
    return (out_mu, out_ls)


# speedup vs baseline: 1.9942x; 1.8498x over previous
import functools

import jax
import jax.numpy as jnp
from jax import lax
from jax.experimental import pallas as pl
from jax.experimental.pallas import tpu as pltpu
from jax.experimental.pallas import tpu_sc as plsc

NC = 2
NS = 16
LANES = 128


def _round_up(a, b):
    return (a + b - 1) // b * b


def _sc_deg(dst3, np_, cpw):
    rps = np_ // NS

    mesh = plsc.VectorSubcoreMesh(core_axis_name="c", subcore_axis_name="s")

    @functools.partial(
        pl.kernel,
        out_type=jax.ShapeDtypeStruct((NC * np_,), jnp.float32),
        mesh=mesh,
        scratch_types=[
            pltpu.VMEM((cpw, LANES), jnp.int32),
            pltpu.VMEM((LANES,), jnp.float32),
            pltpu.VMEM((rps,), jnp.float32),
            pltpu.VMEM_SHARED((np_,), jnp.float32),
            pltpu.SemaphoreType.DMA,
        ],
    )
    def deg_kernel(dst_hbm, degp_hbm, dstv, onesv, zrow, degsh, sem):
        core = lax.axis_index("c")
        sub = lax.axis_index("s")
        wid = sub * NC + core
        pltpu.sync_copy(dst_hbm.at[wid], dstv)

        for k in range(LANES // 16):
            onesv[pl.ds(k * 16, 16)] = jnp.full((16,), 1.0, jnp.float32)

        def zbody(i, _):
            zrow[pl.ds(pl.multiple_of(i * 16, 16), 16)] = jnp.zeros(
                (16,), jnp.float32)
            return 0

        lax.fori_loop(0, rps // 16, zbody, 0)

        base = pl.multiple_of(sub * rps, 128)
        pltpu.sync_copy(zrow, degsh.at[pl.ds(base, rps)])
        plsc.subcore_barrier()

        def ebody(j, _):
            @pl.when(j >= 8)
            def _():
                pltpu.make_async_copy(onesv, degsh.at[dstv.at[j]], sem).wait()

            pltpu.make_async_copy(
                onesv, degsh.at[dstv.at[j]], sem).start(add=True)
            return 0

        lax.fori_loop(0, cpw, ebody, 0)

        def dbody(j, _):
            pltpu.make_async_copy(onesv, degsh.at[dstv.at[j]], sem).wait()
            return 0

        lax.fori_loop(0, min(8, cpw), dbody, 0)
        plsc.subcore_barrier()

        obase = pl.multiple_of(core * np_ + sub * rps, 128)
        pltpu.sync_copy(degsh.at[pl.ds(base, rps)], degp_hbm.at[pl.ds(obase, rps)])

    return deg_kernel(dst3)


def _sc_edges(src3, dst3, xs, dis, np_, cpw, fin):
    rps = np_ // NS
    NR = 2
    ND = 4
    NI = 8
    assert cpw % NI == 0 and cpw >= NI

    mesh = plsc.VectorSubcoreMesh(
        core_axis_name="c", subcore_axis_name="s", num_cores=1)

    @functools.partial(
        pl.kernel,
        out_type=(
            jax.ShapeDtypeStruct((np_, fin), jnp.float32),
            jax.ShapeDtypeStruct((np_,), jnp.float32),
        ),
        mesh=mesh,
        scratch_types=[
            pltpu.VMEM((NI, LANES), jnp.int32),
            pltpu.VMEM((NI, LANES), jnp.int32),
            pltpu.VMEM((NR, LANES, fin), jnp.float32),
            pltpu.VMEM((ND, LANES), jnp.float32),
            pltpu.VMEM((rps,), jnp.float32),
            pltpu.VMEM_SHARED((np_, fin), jnp.float32),
            pltpu.VMEM_SHARED((np_,), jnp.float32),
        ]
        + [pltpu.SemaphoreType.DMA] * (2 * NI + NR + 2 * ND),
    )
    def edge_kernel(src_hbm, dst_hbm, xs_hbm, dis_hbm, tp_hbm, cp_hbm,
                    srcv, dstv, rows, dvals, zrow, tsh, csh, *sems):
        isems = sems[:NI]
        isemd = sems[NI:2 * NI]
        gsem = sems[2 * NI:2 * NI + NR]
        dsem = sems[2 * NI + NR:2 * NI + NR + ND]
        csem = sems[2 * NI + NR + ND:]
        sub = lax.axis_index("s")
        wid = sub

        def idx_start(j, slot):
            pltpu.async_copy(src_hbm.at[wid, j], srcv.at[slot], isems[slot])
            pltpu.async_copy(dst_hbm.at[wid, j], dstv.at[slot], isemd[slot])

        def idx_wait(j, slot):
            pltpu.make_async_copy(
                src_hbm.at[wid, j], srcv.at[slot], isems[slot]).wait()
            pltpu.make_async_copy(
                dst_hbm.at[wid, j], dstv.at[slot], isemd[slot]).wait()

        def rows_start(slot_i, slot_r):
            pltpu.async_copy(
                xs_hbm.at[srcv.at[slot_i]], rows.at[slot_r], gsem[slot_r])

        def rows_wait(slot_i, slot_r):
            pltpu.make_async_copy(
                xs_hbm.at[srcv.at[slot_i]], rows.at[slot_r],
                gsem[slot_r]).wait()

        def dis_start(slot_i, slot_d):
            pltpu.async_copy(
                dis_hbm.at[dstv.at[slot_i]], dvals.at[slot_d], dsem[slot_d])

        def dis_wait(slot_i, slot_d):
            pltpu.make_async_copy(
                dis_hbm.at[dstv.at[slot_i]], dvals.at[slot_d],
                dsem[slot_d]).wait()

        def csc_start(slot_i, slot_d):
            pltpu.make_async_copy(
                dvals.at[slot_d], csh.at[srcv.at[slot_i]],
                csem[slot_d]).start(add=True)

        def csc_wait(slot_i, slot_d):
            pltpu.make_async_copy(
                dvals.at[slot_d], csh.at[srcv.at[slot_i]],
                csem[slot_d]).wait()

        def zr(i, _):
            for k in range(fin // 16):
                rows[0, i, pl.ds(k * 16, 16)] = jnp.zeros((16,), jnp.float32)
            return 0

        lax.fori_loop(0, LANES, zr, 0)

        def zc(i, _):
            zrow[pl.ds(pl.multiple_of(i * 16, 16), 16)] = jnp.zeros(
                (16,), jnp.float32)
            return 0

        lax.fori_loop(0, rps // 16, zc, 0)

        base = pl.multiple_of(sub * rps, 128)
        for k in range(rps // LANES):
            pltpu.sync_copy(rows.at[0], tsh.at[pl.ds(base + k * LANES, LANES)])
        pltpu.sync_copy(zrow, csh.at[pl.ds(base, rps)])

        for j in range(4):
            idx_start(j, j)
        idx_wait(0, 0)
        idx_wait(1, 1)
        rows_start(0, 0)
        dis_start(0, 0)
        dis_start(1, 1)
        plsc.subcore_barrier()

        def ebody(i, _):
            for b in range(NI):
                j = i * NI + b

                @pl.when(j + 4 < cpw)
                def _():
                    idx_start(j + 4, (b + 4) % NI)

                @pl.when(j + 2 < cpw)
                def _():
                    idx_wait(j + 2, (b + 2) % NI)

                @pl.when(j + 1 < cpw)
                def _():
                    rows_start((b + 1) % NI, (b + 1) % NR)

                rows_wait(b, b % NR)
                pltpu.sync_copy(rows.at[b % NR], tsh.at[dstv.at[b]], add=True)

                dis_wait(b, b % ND)
                csc_start(b, b % ND)

                @pl.when(j >= 2)
                def _():
                    csc_wait(b, (b + 2) % ND)

                @pl.when(j + 2 < cpw)
                def _():
                    dis_start((b + 2) % NI, (b + 2) % ND)
            return 0

        lax.fori_loop(0, cpw // NI, ebody, 0)
        csc_wait((cpw - 2) % NI, (cpw - 2) % ND)
        csc_wait((cpw - 1) % NI, (cpw - 1) % ND)
        plsc.subcore_barrier()

        for k in range(rps // LANES):
            pltpu.sync_copy(tsh.at[pl.ds(base + k * LANES, LANES)],
                            tp_hbm.at[pl.ds(base + k * LANES, LANES)])
        pltpu.sync_copy(csh.at[pl.ds(base, rps)], cp_hbm.at[pl.ds(base, rps)])

    return edge_kernel(src3, dst3, xs, dis)


def _tc_matmul(x_pad, w1, np_, fin, bs):
    def body(x_ref, w_ref, o_ref):
        o_ref[:] = jnp.dot(x_ref[:], w_ref[:],
                           preferred_element_type=jnp.float32)

    return pl.pallas_call(
        body,
        grid=(np_ // bs,),
        in_specs=[
            pl.BlockSpec((bs, fin), lambda i: (i, 0)),
            pl.BlockSpec((fin, fin), lambda i: (0, 0)),
        ],
        out_specs=pl.BlockSpec((bs, fin), lambda i: (i, 0)),
        out_shape=jax.ShapeDtypeStruct((np_, fin), jnp.float32),
    )(x_pad, w1)


def _tc_prep(xw, degp, n, np_, fin, bs):
    def body(xw_ref, degp_ref, xs_ref, dis_ref):
        i = pl.program_id(0)
        dp = degp_ref[:]
        degsum = 1.0 + dp[0] + dp[1]
        rows = lax.broadcasted_iota(jnp.int32, (bs, 1), 0) + i * bs
        dis = jnp.where(rows < n, lax.rsqrt(degsum), 0.0)
        xs_ref[:] = xw_ref[:] * dis
        dis_ref[:] = dis

    return pl.pallas_call(
        body,
        grid=(np_ // bs,),
        in_specs=[
            pl.BlockSpec((bs, fin), lambda i: (i, 0)),
            pl.BlockSpec((NC, bs, 1), lambda i: (0, i, 0)),
        ],
        out_specs=[
            pl.BlockSpec((bs, fin), lambda i: (i, 0)),
            pl.BlockSpec((bs, 1), lambda i: (i, 0)),
        ],
        out_shape=(
            jax.ShapeDtypeStruct((np_, fin), jnp.float32),
            jax.ShapeDtypeStruct((np_, 1), jnp.float32),
        ),
    )(xw, degp)


def _tc_gsum(tp0, tp1, xs, dis, cp0, cp1, b1, np_, fin, bs):
    nblk = np_ // bs

    def body(tp0_ref, tp1_ref, xs_ref, dis_ref, cp0_ref, cp1_ref, b1_ref,
             g_ref):
        t = tp0_ref[:] + tp1_ref[:]
        dis = dis_ref[:]
        h = jnp.maximum(dis * (t + xs_ref[:]) + b1_ref[:], 0.0)
        wv = dis * (cp0_ref[:] + cp1_ref[:] + dis)
        g = lax.dot_general(wv, h, (((0,), (0,)), ((), ())),
                            preferred_element_type=jnp.float32)
        g_ref[:] = g.reshape(g_ref.shape)

    return pl.pallas_call(
        body,
        grid=(nblk,),
        in_specs=[
            pl.BlockSpec((bs, fin), lambda i: (i, 0)),
            pl.BlockSpec((bs, fin), lambda i: (i, 0)),
            pl.BlockSpec((bs, fin), lambda i: (i, 0)),
            pl.BlockSpec((bs, 1), lambda i: (i, 0)),
            pl.BlockSpec((bs, 1), lambda i: (i, 0)),
            pl.BlockSpec((bs, 1), lambda i: (i, 0)),
            pl.BlockSpec((1, fin), lambda i: (0, 0)),
        ],
        out_specs=pl.BlockSpec((1, 1, fin), lambda i: (i, 0, 0)),
        out_shape=jax.ShapeDtypeStruct((nblk, 1, fin), jnp.float32),
    )(tp0, tp1, xs, dis, cp0, cp1, b1)


def _tc_heads(gparts, wmu, bmu, wls, bls, n, fin, fout, bs):
    nblk = n // bs
    inv_n = 1.0 / n

    def body(g_ref, wmu_ref, bmu_ref, wls_ref, bls_ref, omu_ref, ols_ref):
        g = jnp.sum(g_ref[:], axis=0, keepdims=True) * inv_n
        mu = jnp.dot(g, wmu_ref[:], preferred_element_type=jnp.float32) \
            + bmu_ref[:]
        ls = jnp.dot(g, wls_ref[:], preferred_element_type=jnp.float32) \
            + bls_ref[:]
        omu_ref[:] = jnp.broadcast_to(mu, (bs, mu.shape[1]))
        ols_ref[:] = jnp.broadcast_to(ls, (bs, ls.shape[1]))

    nparts = gparts.shape[0]
    return pl.pallas_call(
        body,
        grid=(nblk,),
        in_specs=[
            pl.BlockSpec((nparts, fin), lambda i: (0, 0)),
            pl.BlockSpec((fin, fout), lambda i: (0, 0)),
            pl.BlockSpec((1, fout), lambda i: (0, 0)),
            pl.BlockSpec((fin, fout), lambda i: (0, 0)),
            pl.BlockSpec((1, fout), lambda i: (0, 0)),
        ],
        out_specs=[
            pl.BlockSpec((bs, fout), lambda i: (i, 0)),
            pl.BlockSpec((bs, fout), lambda i: (i, 0)),
        ],
        out_shape=(
            jax.ShapeDtypeStruct((n, fout), jnp.float32),
            jax.ShapeDtypeStruct((n, fout), jnp.float32),
        ),
    )(gparts, wmu, bmu, wls, bls)


def kernel(x, edge_index, W1, b1, Wmu, bmu, Wls, bls):
    n, fin = x.shape
    e = edge_index.shape[1]
    fout = Wmu.shape[1]
    nw = NC * NS

    np_ = _round_up(n + 1, NS * LANES)
    ep = _round_up(e, nw * LANES * 8)
    cpw = ep // (nw * LANES)

    src = edge_index[0]
    dst = edge_index[1]
    pad_e = ep - e
    pad_i = lax.iota(jnp.int32, pad_e)
    src_p = jnp.concatenate(
        [src, pad_i % n]).reshape(2, NS, cpw, LANES)
    dst_p = jnp.concatenate(
        [dst, n + pad_i % (np_ - n)]).reshape(2, NS, cpw, LANES)

    x_pad = jnp.pad(x, ((0, np_ - n), (0, 0)))

    degp = _sc_deg(dst_p.reshape(nw, cpw, LANES), np_, cpw)
    xw = _tc_matmul(x_pad, W1, np_, fin, 1024)

    degp2 = degp.reshape(NC, np_, 1)
    xs, dis2 = _tc_prep(xw, degp2, n, np_, fin, 1024)

    dis1 = dis2.reshape(np_)
    tp0, cp0 = _sc_edges(src_p[0], dst_p[0], xs, dis1, np_, cpw, fin)
    tp1, cp1 = _sc_edges(src_p[1], dst_p[1], xs, dis1, np_, cpw, fin)

    gparts = _tc_gsum(tp0, tp1, xs, dis2,
                      cp0.reshape(np_, 1), cp1.reshape(np_, 1),
                      b1.reshape(1, fin),
                      np_, fin, 1024).reshape(-1, fin)

    out_mu, out_ls = _tc_heads(gparts, Wmu, bmu.reshape(1, fout),
                               Wls, bls.reshape(1, fout), n, fin, fout, 1000)
    return (out_mu, out_ls)


# speedup vs baseline: 2.9423x; 1.4754x over previous
import functools

import jax
import jax.numpy as jnp
from jax import lax
from jax.experimental import pallas as pl
from jax.experimental.pallas import tpu as pltpu
from jax.experimental.pallas import tpu_sc as plsc

NC = 2
NS = 16
LANES = 128


def _round_up(a, b):
    return (a + b - 1) // b * b


def _sc_deg(dst3, np_, cpw):
    rps = np_ // NS

    mesh = plsc.VectorSubcoreMesh(core_axis_name="c", subcore_axis_name="s")

    @functools.partial(
        pl.kernel,
        out_type=jax.ShapeDtypeStruct((NC * np_,), jnp.float32),
        mesh=mesh,
        scratch_types=[
            pltpu.VMEM((cpw, LANES), jnp.int32),
            pltpu.VMEM((LANES,), jnp.float32),
            pltpu.VMEM((rps,), jnp.float32),
            pltpu.VMEM_SHARED((np_,), jnp.float32),
            pltpu.SemaphoreType.DMA,
        ],
    )
    def deg_kernel(dst_hbm, degp_hbm, dstv, onesv, zrow, degsh, sem):
        core = lax.axis_index("c")
        sub = lax.axis_index("s")
        wid = sub * NC + core
        pltpu.sync_copy(dst_hbm.at[wid], dstv)

        for k in range(LANES // 16):
            onesv[pl.ds(k * 16, 16)] = jnp.full((16,), 1.0, jnp.float32)

        def zbody(i, _):
            zrow[pl.ds(pl.multiple_of(i * 16, 16), 16)] = jnp.zeros(
                (16,), jnp.float32)
            return 0

        lax.fori_loop(0, rps // 16, zbody, 0)

        base = pl.multiple_of(sub * rps, 128)
        pltpu.sync_copy(zrow, degsh.at[pl.ds(base, rps)])
        plsc.subcore_barrier()

        def ebody(j, _):
            @pl.when(j >= 8)
            def _():
                pltpu.make_async_copy(onesv, degsh.at[dstv.at[j]], sem).wait()

            pltpu.make_async_copy(
                onesv, degsh.at[dstv.at[j]], sem).start(add=True)
            return 0

        lax.fori_loop(0, cpw, ebody, 0)

        def dbody(j, _):
            pltpu.make_async_copy(onesv, degsh.at[dstv.at[j]], sem).wait()
            return 0

        lax.fori_loop(0, min(8, cpw), dbody, 0)
        plsc.subcore_barrier()

        obase = pl.multiple_of(core * np_ + sub * rps, 128)
        pltpu.sync_copy(degsh.at[pl.ds(base, rps)], degp_hbm.at[pl.ds(obase, rps)])

    return deg_kernel(dst3)


def _sc_edges(src3, dst3, xs, dis, np_, cpw, fin):
    rps = np_ // NS
    NR = 2
    ND = 4
    NI = 8
    assert cpw % NI == 0 and cpw >= NI

    mesh = plsc.VectorSubcoreMesh(core_axis_name="c", subcore_axis_name="s")

    @functools.partial(
        pl.kernel,
        out_type=(
            jax.ShapeDtypeStruct((NC * np_, fin), jnp.float32),
            jax.ShapeDtypeStruct((NC * np_,), jnp.float32),
        ),
        mesh=mesh,
        scratch_types=[
            pltpu.VMEM((NI, LANES), jnp.int32),
            pltpu.VMEM((NI, LANES), jnp.int32),
            pltpu.VMEM((NR, LANES, fin), jnp.float32),
            pltpu.VMEM((ND, LANES), jnp.float32),
            pltpu.VMEM((rps,), jnp.float32),
            pltpu.VMEM_SHARED((np_, fin), jnp.float32),
            pltpu.VMEM_SHARED((np_,), jnp.float32),
        ]
        + [pltpu.SemaphoreType.DMA] * (2 * NI + NR + 2 * ND),
    )
    def edge_kernel(src_hbm, dst_hbm, xs_hbm, dis_hbm, tp_hbm, cp_hbm,
                    srcv, dstv, rows, dvals, zrow, tsh, csh, *sems):
        isems = sems[:NI]
        isemd = sems[NI:2 * NI]
        gsem = sems[2 * NI:2 * NI + NR]
        dsem = sems[2 * NI + NR:2 * NI + NR + ND]
        csem = sems[2 * NI + NR + ND:]
        core = lax.axis_index("c")
        sub = lax.axis_index("s")
        wid = sub * NC + core

        def idx_start(j, slot):
            pltpu.async_copy(src_hbm.at[wid, j], srcv.at[slot], isems[slot])
            pltpu.async_copy(dst_hbm.at[wid, j], dstv.at[slot], isemd[slot])

        def idx_wait(j, slot):
            pltpu.make_async_copy(
                src_hbm.at[wid, j], srcv.at[slot], isems[slot]).wait()
            pltpu.make_async_copy(
                dst_hbm.at[wid, j], dstv.at[slot], isemd[slot]).wait()

        def rows_start(slot_i, slot_r):
            pltpu.async_copy(
                xs_hbm.at[srcv.at[slot_i]], rows.at[slot_r], gsem[slot_r])

        def rows_wait(slot_i, slot_r):
            pltpu.make_async_copy(
                xs_hbm.at[srcv.at[slot_i]], rows.at[slot_r],
                gsem[slot_r]).wait()

        def dis_start(slot_i, slot_d):
            pltpu.async_copy(
                dis_hbm.at[dstv.at[slot_i]], dvals.at[slot_d], dsem[slot_d])

        def dis_wait(slot_i, slot_d):
            pltpu.make_async_copy(
                dis_hbm.at[dstv.at[slot_i]], dvals.at[slot_d],
                dsem[slot_d]).wait()

        def csc_start(slot_i, slot_d):
            pltpu.make_async_copy(
                dvals.at[slot_d], csh.at[srcv.at[slot_i]],
                csem[slot_d]).start(add=True)

        def csc_wait(slot_i, slot_d):
            pltpu.make_async_copy(
                dvals.at[slot_d], csh.at[srcv.at[slot_i]],
                csem[slot_d]).wait()

        def zr(i, _):
            for k in range(fin // 16):
                rows[0, i, pl.ds(k * 16, 16)] = jnp.zeros((16,), jnp.float32)
            return 0

        lax.fori_loop(0, LANES, zr, 0)

        def zc(i, _):
            zrow[pl.ds(pl.multiple_of(i * 16, 16), 16)] = jnp.zeros(
                (16,), jnp.float32)
            return 0

        lax.fori_loop(0, rps // 16, zc, 0)

        base = pl.multiple_of(sub * rps, 128)
        for k in range(rps // LANES):
            pltpu.sync_copy(rows.at[0], tsh.at[pl.ds(base + k * LANES, LANES)])
        pltpu.sync_copy(zrow, csh.at[pl.ds(base, rps)])

        for j in range(4):
            idx_start(j, j)
        idx_wait(0, 0)
        idx_wait(1, 1)
        rows_start(0, 0)
        dis_start(0, 0)
        dis_start(1, 1)
        plsc.subcore_barrier()

        def ebody(i, _):
            for b in range(NI):
                j = i * NI + b

                @pl.when(j + 4 < cpw)
                def _():
                    idx_start(j + 4, (b + 4) % NI)

                @pl.when(j + 2 < cpw)
                def _():
                    idx_wait(j + 2, (b + 2) % NI)

                @pl.when(j + 1 < cpw)
                def _():
                    rows_start((b + 1) % NI, (b + 1) % NR)

                rows_wait(b, b % NR)
                pltpu.sync_copy(rows.at[b % NR], tsh.at[dstv.at[b]], add=True)

                dis_wait(b, b % ND)
                csc_start(b, b % ND)

                @pl.when(j >= 2)
                def _():
                    csc_wait(b, (b + 2) % ND)

                @pl.when(j + 2 < cpw)
                def _():
                    dis_start((b + 2) % NI, (b + 2) % ND)
            return 0

        lax.fori_loop(0, cpw // NI, ebody, 0)
        csc_wait((cpw - 2) % NI, (cpw - 2) % ND)
        csc_wait((cpw - 1) % NI, (cpw - 1) % ND)
        plsc.subcore_barrier()

        obase = pl.multiple_of(core * np_ + sub * rps, 128)
        for k in range(rps // LANES):
            pltpu.sync_copy(tsh.at[pl.ds(base + k * LANES, LANES)],
                            tp_hbm.at[pl.ds(obase + k * LANES, LANES)])
        pltpu.sync_copy(csh.at[pl.ds(base, rps)], cp_hbm.at[pl.ds(obase, rps)])

    return edge_kernel(src3, dst3, xs, dis)


def _tc_matmul(x_pad, w1, np_, fin, bs):
    def body(x_ref, w_ref, o_ref):
        o_ref[:] = jnp.dot(x_ref[:], w_ref[:],
                           preferred_element_type=jnp.float32)

    return pl.pallas_call(
        body,
        grid=(np_ // bs,),
        in_specs=[
            pl.BlockSpec((bs, fin), lambda i: (i, 0)),
            pl.BlockSpec((fin, fin), lambda i: (0, 0)),
        ],
        out_specs=pl.BlockSpec((bs, fin), lambda i: (i, 0)),
        out_shape=jax.ShapeDtypeStruct((np_, fin), jnp.float32),
    )(x_pad, w1)


def _tc_prep(xw, degp, n, np_, fin, bs):
    def body(xw_ref, degp_ref, xs_ref, dis_ref):
        i = pl.program_id(0)
        dp = degp_ref[:]
        degsum = 1.0 + dp[0] + dp[1]
        rows = lax.broadcasted_iota(jnp.int32, (bs, 1), 0) + i * bs
        dis = jnp.where(rows < n, lax.rsqrt(degsum), 0.0)
        xs_ref[:] = xw_ref[:] * dis
        dis_ref[:] = dis

    return pl.pallas_call(
        body,
        grid=(np_ // bs,),
        in_specs=[
            pl.BlockSpec((bs, fin), lambda i: (i, 0)),
            pl.BlockSpec((NC, bs, 1), lambda i: (0, i, 0)),
        ],
        out_specs=[
            pl.BlockSpec((bs, fin), lambda i: (i, 0)),
            pl.BlockSpec((bs, 1), lambda i: (i, 0)),
        ],
        out_shape=(
            jax.ShapeDtypeStruct((np_, fin), jnp.float32),
            jax.ShapeDtypeStruct((np_, 1), jnp.float32),
        ),
    )(xw, degp)


def _tc_gsum(tp0, tp1, xs, dis, cp0, cp1, b1, np_, fin, bs):
    nblk = np_ // bs

    def body(tp0_ref, tp1_ref, xs_ref, dis_ref, cp0_ref, cp1_ref, b1_ref,
             g_ref):
        t = tp0_ref[:] + tp1_ref[:]
        dis = dis_ref[:]
        h = jnp.maximum(dis * (t + xs_ref[:]) + b1_ref[:], 0.0)
        wv = dis * (cp0_ref[:] + cp1_ref[:] + dis)
        g = lax.dot_general(wv, h, (((0,), (0,)), ((), ())),
                            preferred_element_type=jnp.float32)
        g_ref[:] = g.reshape(g_ref.shape)

    return pl.pallas_call(
        body,
        grid=(nblk,),
        in_specs=[
            pl.BlockSpec((bs, fin), lambda i: (i, 0)),
            pl.BlockSpec((bs, fin), lambda i: (nblk + i, 0)),
            pl.BlockSpec((bs, fin), lambda i: (i, 0)),
            pl.BlockSpec((bs, 1), lambda i: (i, 0)),
            pl.BlockSpec((bs, 1), lambda i: (i, 0)),
            pl.BlockSpec((bs, 1), lambda i: (nblk + i, 0)),
            pl.BlockSpec((1, fin), lambda i: (0, 0)),
        ],
        out_specs=pl.BlockSpec((1, 1, fin), lambda i: (i, 0, 0)),
        out_shape=jax.ShapeDtypeStruct((nblk, 1, fin), jnp.float32),
    )(tp0, tp1, xs, dis, cp0, cp1, b1)


def _tc_heads(gparts, wmu, bmu, wls, bls, n, fin, fout, bs):
    nblk = n // bs
    inv_n = 1.0 / n

    def body(g_ref, wmu_ref, bmu_ref, wls_ref, bls_ref, omu_ref, ols_ref):
        g = jnp.sum(g_ref[:], axis=0, keepdims=True) * inv_n
        mu = jnp.dot(g, wmu_ref[:], preferred_element_type=jnp.float32) \
            + bmu_ref[:]
        ls = jnp.dot(g, wls_ref[:], preferred_element_type=jnp.float32) \
            + bls_ref[:]
        omu_ref[:] = jnp.broadcast_to(mu, (bs, mu.shape[1]))
        ols_ref[:] = jnp.broadcast_to(ls, (bs, ls.shape[1]))

    nparts = gparts.shape[0]
    return pl.pallas_call(
        body,
        grid=(nblk,),
        in_specs=[
            pl.BlockSpec((nparts, fin), lambda i: (0, 0)),
            pl.BlockSpec((fin, fout), lambda i: (0, 0)),
            pl.BlockSpec((1, fout), lambda i: (0, 0)),
            pl.BlockSpec((fin, fout), lambda i: (0, 0)),
            pl.BlockSpec((1, fout), lambda i: (0, 0)),
        ],
        out_specs=[
            pl.BlockSpec((bs, fout), lambda i: (i, 0)),
            pl.BlockSpec((bs, fout), lambda i: (i, 0)),
        ],
        out_shape=(
            jax.ShapeDtypeStruct((n, fout), jnp.float32),
            jax.ShapeDtypeStruct((n, fout), jnp.float32),
        ),
    )(gparts, wmu, bmu, wls, bls)


def kernel(x, edge_index, W1, b1, Wmu, bmu, Wls, bls):
    n, fin = x.shape
    e = edge_index.shape[1]
    fout = Wmu.shape[1]
    nw = NC * NS

    np_ = _round_up(n + 1, NS * LANES)
    ep = _round_up(e, nw * LANES * 8)
    cpw = ep // (nw * LANES)

    src = edge_index[0]
    dst = edge_index[1]
    pad_e = ep - e
    pad_i = lax.iota(jnp.int32, pad_e)
    src_p = jnp.concatenate(
        [src, pad_i % n]).reshape(nw, cpw, LANES)
    dst_p = jnp.concatenate(
        [dst, n + pad_i % (np_ - n)]).reshape(nw, cpw, LANES)

    x_pad = jnp.pad(x, ((0, np_ - n), (0, 0)))

    degp = _sc_deg(dst_p, np_, cpw)
    xw = _tc_matmul(x_pad, W1, np_, fin, 1024)

    degp2 = degp.reshape(NC, np_, 1)
    xs, dis2 = _tc_prep(xw, degp2, n, np_, fin, 1024)

    dis1 = dis2.reshape(np_)
    tp, cp = _sc_edges(src_p, dst_p, xs, dis1, np_, cpw, fin)

    gparts = _tc_gsum(tp, tp, xs, dis2,
                      cp.reshape(NC * np_, 1), cp.reshape(NC * np_, 1),
                      b1.reshape(1, fin),
                      np_, fin, 1024).reshape(-1, fin)

    out_mu, out_ls = _tc_heads(gparts, Wmu, bmu.reshape(1, fout),
                               Wls, bls.reshape(1, fout), n, fin, fout, 1000)
    return (out_mu, out_ls)


# speedup vs baseline: 3.6532x; 1.2416x over previous
import functools

import jax
import jax.numpy as jnp
from jax import lax
from jax.experimental import pallas as pl
from jax.experimental.pallas import tpu as pltpu
from jax.experimental.pallas import tpu_sc as plsc

NC = 2
NS = 16
LANES = 128


def _round_up(a, b):
    return (a + b - 1) // b * b


def _sc_deg(dst3, np_, cpw):
    rps = np_ // NS

    mesh = plsc.VectorSubcoreMesh(core_axis_name="c", subcore_axis_name="s")

    @functools.partial(
        pl.kernel,
        out_type=jax.ShapeDtypeStruct((NC * np_,), jnp.float32),
        mesh=mesh,
        scratch_types=[
            pltpu.VMEM((cpw, LANES), jnp.int32),
            pltpu.VMEM((LANES,), jnp.float32),
            pltpu.VMEM((rps,), jnp.float32),
            pltpu.VMEM_SHARED((np_,), jnp.float32),
            pltpu.SemaphoreType.DMA,
        ],
    )
    def deg_kernel(dst_hbm, degp_hbm, dstv, onesv, zrow, degsh, sem):
        core = lax.axis_index("c")
        sub = lax.axis_index("s")
        wid = sub * NC + core
        pltpu.sync_copy(dst_hbm.at[wid], dstv)

        for k in range(LANES // 16):
            onesv[pl.ds(k * 16, 16)] = jnp.full((16,), 1.0, jnp.float32)

        def zbody(i, _):
            zrow[pl.ds(pl.multiple_of(i * 16, 16), 16)] = jnp.zeros(
                (16,), jnp.float32)
            return 0

        lax.fori_loop(0, rps // 16, zbody, 0)

        base = pl.multiple_of(sub * rps, 128)
        pltpu.sync_copy(zrow, degsh.at[pl.ds(base, rps)])
        plsc.subcore_barrier()

        def ebody(j, _):
            @pl.when(j >= 8)
            def _():
                pltpu.make_async_copy(onesv, degsh.at[dstv.at[j]], sem).wait()

            pltpu.make_async_copy(
                onesv, degsh.at[dstv.at[j]], sem).start(add=True)
            return 0

        lax.fori_loop(0, cpw, ebody, 0)

        def dbody(j, _):
            pltpu.make_async_copy(onesv, degsh.at[dstv.at[j]], sem).wait()
            return 0

        lax.fori_loop(0, min(8, cpw), dbody, 0)
        plsc.subcore_barrier()

        obase = pl.multiple_of(core * np_ + sub * rps, 128)
        pltpu.sync_copy(degsh.at[pl.ds(base, rps)], degp_hbm.at[pl.ds(obase, rps)])

    return deg_kernel(dst3)


def _sc_edges(src3, dst3, xs, dis, np_, cpw, fin):
    rps = np_ // NS
    NR = 2
    ND = 4
    NI = 8
    assert cpw % NI == 0 and cpw >= NI

    mesh = plsc.VectorSubcoreMesh(core_axis_name="c", subcore_axis_name="s")

    @functools.partial(
        pl.kernel,
        out_type=(
            jax.ShapeDtypeStruct((NC * np_, fin), jnp.float32),
            jax.ShapeDtypeStruct((NC * np_,), jnp.float32),
        ),
        mesh=mesh,
        scratch_types=[
            pltpu.VMEM((NI, LANES), jnp.int32),
            pltpu.VMEM((NI, LANES), jnp.int32),
            pltpu.VMEM((NR, LANES, fin), jnp.float32),
            pltpu.VMEM((ND, LANES), jnp.float32),
            pltpu.VMEM((rps,), jnp.float32),
            pltpu.VMEM_SHARED((np_, fin), jnp.float32),
            pltpu.VMEM_SHARED((np_,), jnp.float32),
        ]
        + [pltpu.SemaphoreType.DMA] * (2 * NI + NR + 2 * ND),
    )
    def edge_kernel(src_hbm, dst_hbm, xs_hbm, dis_hbm, tp_hbm, cp_hbm,
                    srcv, dstv, rows, dvals, zrow, tsh, csh, *sems):
        isems = sems[:NI]
        isemd = sems[NI:2 * NI]
        gsem = sems[2 * NI:2 * NI + NR]
        dsem = sems[2 * NI + NR:2 * NI + NR + ND]
        csem = sems[2 * NI + NR + ND:]
        core = lax.axis_index("c")
        sub = lax.axis_index("s")
        wid = sub * NC + core

        def idx_start(j, slot):
            pltpu.async_copy(src_hbm.at[wid, j], srcv.at[slot], isems[slot])
            pltpu.async_copy(dst_hbm.at[wid, j], dstv.at[slot], isemd[slot])

        def idx_wait(j, slot):
            pltpu.make_async_copy(
                src_hbm.at[wid, j], srcv.at[slot], isems[slot]).wait()
            pltpu.make_async_copy(
                dst_hbm.at[wid, j], dstv.at[slot], isemd[slot]).wait()

        def rows_start(slot_i, slot_r):
            pltpu.async_copy(
                xs_hbm.at[srcv.at[slot_i]], rows.at[slot_r], gsem[slot_r])

        def rows_wait(slot_i, slot_r):
            pltpu.make_async_copy(
                xs_hbm.at[srcv.at[slot_i]], rows.at[slot_r],
                gsem[slot_r]).wait()

        def dis_start(slot_i, slot_d):
            pltpu.async_copy(
                dis_hbm.at[dstv.at[slot_i]], dvals.at[slot_d], dsem[slot_d])

        def dis_wait(slot_i, slot_d):
            pltpu.make_async_copy(
                dis_hbm.at[dstv.at[slot_i]], dvals.at[slot_d],
                dsem[slot_d]).wait()

        def csc_start(slot_i, slot_d):
            pltpu.make_async_copy(
                dvals.at[slot_d], csh.at[srcv.at[slot_i]],
                csem[slot_d]).start(add=True)

        def csc_wait(slot_i, slot_d):
            pltpu.make_async_copy(
                dvals.at[slot_d], csh.at[srcv.at[slot_i]],
                csem[slot_d]).wait()

        def zr(i, _):
            for k in range(fin // 16):
                rows[0, i, pl.ds(k * 16, 16)] = jnp.zeros((16,), jnp.float32)
            return 0

        lax.fori_loop(0, LANES, zr, 0)

        def zc(i, _):
            zrow[pl.ds(pl.multiple_of(i * 16, 16), 16)] = jnp.zeros(
                (16,), jnp.float32)
            return 0

        lax.fori_loop(0, rps // 16, zc, 0)

        base = pl.multiple_of(sub * rps, 128)
        for k in range(rps // LANES):
            pltpu.sync_copy(rows.at[0], tsh.at[pl.ds(base + k * LANES, LANES)])
        pltpu.sync_copy(zrow, csh.at[pl.ds(base, rps)])

        for j in range(4):
            idx_start(j, j)
        idx_wait(0, 0)
        idx_wait(1, 1)
        rows_start(0, 0)
        dis_start(0, 0)
        dis_start(1, 1)
        plsc.subcore_barrier()

        def ebody(i, _):
            for b in range(NI):
                j = i * NI + b

                @pl.when(j + 4 < cpw)
                def _():
                    idx_start(j + 4, (b + 4) % NI)

                @pl.when(j + 2 < cpw)
                def _():
                    idx_wait(j + 2, (b + 2) % NI)

                @pl.when(j + 1 < cpw)
                def _():
                    rows_start((b + 1) % NI, (b + 1) % NR)

                rows_wait(b, b % NR)
                pltpu.sync_copy(rows.at[b % NR], tsh.at[dstv.at[b]], add=True)

                dis_wait(b, b % ND)
                csc_start(b, b % ND)

                @pl.when(j >= 2)
                def _():
                    csc_wait(b, (b + 2) % ND)

                @pl.when(j + 2 < cpw)
                def _():
                    dis_start((b + 2) % NI, (b + 2) % ND)
            return 0

        lax.fori_loop(0, cpw // NI, ebody, 0)
        csc_wait((cpw - 2) % NI, (cpw - 2) % ND)
        csc_wait((cpw - 1) % NI, (cpw - 1) % ND)
        plsc.subcore_barrier()

        obase = pl.multiple_of(core * np_ + sub * rps, 128)
        for k in range(rps // LANES):
            pltpu.sync_copy(tsh.at[pl.ds(base + k * LANES, LANES)],
                            tp_hbm.at[pl.ds(obase + k * LANES, LANES)])
        pltpu.sync_copy(csh.at[pl.ds(base, rps)], cp_hbm.at[pl.ds(obase, rps)])

    return edge_kernel(src3, dst3, xs, dis)


def _tc_prep(x_pad, w1, degp2d, n, np_, fin, bs):
    spb = bs // 128
    prows = np_ // 128

    def body(x_ref, w_ref, dp0_ref, dp1_ref, xs_ref, dis_ref):
        i = pl.program_id(0)
        degsum = 1.0 + dp0_ref[:] + dp1_ref[:]
        rows = (i * bs
                + lax.broadcasted_iota(jnp.int32, (spb, 128), 0) * 128
                + lax.broadcasted_iota(jnp.int32, (spb, 128), 1))
        dis = jnp.where(rows < n, lax.rsqrt(degsum), 0.0)
        xw = jnp.dot(x_ref[:], w_ref[:], preferred_element_type=jnp.float32)
        for s in range(spb):
            dcol = jnp.transpose(dis[s:s + 1, :])
            xs_ref[pl.ds(s * 128, 128), :] = \
                xw[s * 128:(s + 1) * 128, :] * dcol
        dis_ref[:] = dis

    return pl.pallas_call(
        body,
        grid=(np_ // bs,),
        in_specs=[
            pl.BlockSpec((bs, fin), lambda i: (i, 0)),
            pl.BlockSpec((fin, fin), lambda i: (0, 0)),
            pl.BlockSpec((spb, 128), lambda i: (i, 0)),
            pl.BlockSpec((spb, 128), lambda i, _p=prows // spb: (_p + i, 0)),
        ],
        out_specs=[
            pl.BlockSpec((bs, fin), lambda i: (i, 0)),
            pl.BlockSpec((spb, 128), lambda i: (i, 0)),
        ],
        out_shape=(
            jax.ShapeDtypeStruct((np_, fin), jnp.float32),
            jax.ShapeDtypeStruct((prows, 128), jnp.float32),
        ),
    )(x_pad, w1, degp2d, degp2d)


def _tc_gsum_heads(tp, xs, dis2d, cp2d, b1, wmu, bmu, wls, bls,
                   n, np_, fin, fout, bs):
    nblk = np_ // bs
    spb = bs // 128
    prows = (NC * np_) // 128
    inv_n = 1.0 / n

    def body(tp0_ref, tp1_ref, xs_ref, dis_ref, cp0_ref, cp1_ref, b1_ref,
             wmu_ref, bmu_ref, wls_ref, bls_ref, omu_ref, ols_ref, gacc):
        i = pl.program_id(0)
        t = tp0_ref[:] + tp1_ref[:]
        dis = dis_ref[:]
        wv = dis * (cp0_ref[:] + cp1_ref[:] + dis)
        xs = xs_ref[:]
        b1row = b1_ref[:]
        g = jnp.zeros((1, fin), jnp.float32)
        for s in range(spb):
            dcol = jnp.transpose(dis[s:s + 1, :])
            h_s = jnp.maximum(
                dcol * (t[s * 128:(s + 1) * 128, :]
                        + xs[s * 128:(s + 1) * 128, :]) + b1row, 0.0)
            g = g + lax.dot_general(
                wv[s:s + 1, :], h_s, (((1,), (0,)), ((), ())),
                preferred_element_type=jnp.float32)

        @pl.when(i == 0)
        def _():
            gacc[:] = g

        @pl.when(i > 0)
        def _():
            gacc[:] = gacc[:] + g

        @pl.when(i == nblk - 1)
        def _():
            gn = gacc[:] * inv_n
            omu_ref[:] = jnp.dot(gn, wmu_ref[:],
                                 preferred_element_type=jnp.float32) \
                + bmu_ref[:]
            ols_ref[:] = jnp.dot(gn, wls_ref[:],
                                 preferred_element_type=jnp.float32) \
                + bls_ref[:]

    return pl.pallas_call(
        body,
        grid=(nblk,),
        in_specs=[
            pl.BlockSpec((bs, fin), lambda i: (i, 0)),
            pl.BlockSpec((bs, fin), lambda i: (nblk + i, 0)),
            pl.BlockSpec((bs, fin), lambda i: (i, 0)),
            pl.BlockSpec((spb, 128), lambda i: (i, 0)),
            pl.BlockSpec((spb, 128), lambda i: (i, 0)),
            pl.BlockSpec((spb, 128), lambda i, _p=prows // (2 * spb):
                         (_p + i, 0)),
            pl.BlockSpec((1, fin), lambda i: (0, 0)),
            pl.BlockSpec((fin, fout), lambda i: (0, 0)),
            pl.BlockSpec((1, fout), lambda i: (0, 0)),
            pl.BlockSpec((fin, fout), lambda i: (0, 0)),
            pl.BlockSpec((1, fout), lambda i: (0, 0)),
        ],
        out_specs=[
            pl.BlockSpec((1, fout), lambda i: (0, 0)),
            pl.BlockSpec((1, fout), lambda i: (0, 0)),
        ],
        out_shape=(
            jax.ShapeDtypeStruct((1, fout), jnp.float32),
            jax.ShapeDtypeStruct((1, fout), jnp.float32),
        ),
        scratch_shapes=[pltpu.VMEM((1, fin), jnp.float32)],
    )(tp, tp, xs, dis2d, cp2d, cp2d, b1, wmu, bmu, wls, bls)


def kernel(x, edge_index, W1, b1, Wmu, bmu, Wls, bls):
    n, fin = x.shape
    e = edge_index.shape[1]
    fout = Wmu.shape[1]
    nw = NC * NS

    np_ = _round_up(n + 1, NS * LANES)
    ep = _round_up(e, nw * LANES * 8)
    cpw = ep // (nw * LANES)

    src = edge_index[0]
    dst = edge_index[1]
    pad_e = ep - e
    pad_i = lax.iota(jnp.int32, pad_e)
    src_p = jnp.concatenate(
        [src, pad_i % n]).reshape(nw, cpw, LANES)
    dst_p = jnp.concatenate(
        [dst, n + pad_i % (np_ - n)]).reshape(nw, cpw, LANES)

    x_pad = jnp.pad(x, ((0, np_ - n), (0, 0)))

    degp = _sc_deg(dst_p, np_, cpw)

    degp2d = degp.reshape((NC * np_) // 128, 128)
    xs, dis2d = _tc_prep(x_pad, W1, degp2d, n, np_, fin, 1024)

    tp, cp = _sc_edges(src_p, dst_p, xs, dis2d.reshape(np_), np_, cpw, fin)

    mu1, ls1 = _tc_gsum_heads(tp, xs, dis2d,
                              cp.reshape((NC * np_) // 128, 128),
                              b1.reshape(1, fin), Wmu, bmu.reshape(1, fout),
                              Wls, bls.reshape(1, fout),
                              n, np_, fin, fout, 1024)
    return (jnp.broadcast_to(mu1, (n, fout)), jnp.broadcast_to(ls1, (n, fout)))
